# baseline (speedup 1.0000x reference)
import jax
import jax.numpy as jnp
from jax.experimental import pallas as pl
from jax.experimental.pallas import tpu as pltpu


def _fused_kernel(q_ref, k_ref, v_ref, o_ref, qs_ref, s_ref, mm_ref):
    j = pl.program_id(1)
    sblk, h, e, p = v_ref.shape[1:]

    @pl.when(j == 0)
    def _qsum():
        qs_ref[...] = jnp.sum(q_ref[0, 0], axis=-1, keepdims=True)

    @pl.when(j < 2)
    def _scores():
        sb = jnp.sum(k_ref[0] * qs_ref[...][None], axis=2)
        s_ref[pl.ds(j * sblk, sblk)] = sb
        mn = jnp.min(sb, axis=(0, 2), keepdims=True)[0]
        mx = jnp.max(sb, axis=(0, 2), keepdims=True)[0]

        @pl.when(j == 0)
        def _init():
            mm_ref[0] = mn
            mm_ref[1] = mx

        @pl.when(j > 0)
        def _update():
            mm_ref[0] = jnp.minimum(mm_ref[0], mn)
            mm_ref[1] = jnp.maximum(mm_ref[1], mx)

    @pl.when(j >= 2)
    def _weight():
        mn = mm_ref[0]
        mx = mm_ref[1]
        inv = 1.0 / (mx - mn)
        attn = (s_ref[pl.ds((j - 2) * sblk, sblk)] - mn) * inv
        o_ref[0] = v_ref[0] * attn.reshape(sblk, h, 1, p)


@jax.jit
def kernel(queries, keys, values, attn_mask):
    del attn_mask
    B, L, Qd, H, E = queries.shape
    S, P = keys.shape[1], keys.shape[2]
    SB = 32
    NP = S // SB

    qt = jnp.transpose(queries, (0, 2, 3, 4, 1))
    kt = jnp.transpose(keys, (0, 1, 3, 4, 2))
    vt = jnp.transpose(values, (0, 1, 3, 4, 2))

    out_t = pl.pallas_call(
        _fused_kernel,
        grid=(B, 2 * NP),
        in_specs=[
            pl.BlockSpec((1, 1, H, E, L), lambda b, j: (b, Qd - 1, 0, 0, 0)),
            pl.BlockSpec((1, SB, H, E, P),
                         lambda b, j: (b, jnp.minimum(j, 1), 0, 0, 0)),
            pl.BlockSpec((1, SB, H, E, P),
                         lambda b, j: (b, jnp.maximum(j - 2, 0), 0, 0, 0)),
        ],
        out_specs=pl.BlockSpec((1, SB, H, E, P),
                               lambda b, j: (b, jnp.maximum(j - 2, 0), 0, 0, 0)),
        out_shape=jax.ShapeDtypeStruct((B, S, H, E, P), jnp.float32),
        scratch_shapes=[
            pltpu.VMEM((H, E, 1), jnp.float32),
            pltpu.VMEM((S, H, P), jnp.float32),
            pltpu.VMEM((2, H, 1), jnp.float32),
        ],
        compiler_params=pltpu.CompilerParams(
            dimension_semantics=("parallel", "arbitrary"),
            vmem_limit_bytes=58 * 1024 * 1024),
    )(qt, kt, vt)

    return jnp.transpose(out_t, (0, 1, 4, 2, 3))


# speedup vs baseline: 1.1274x; 1.1274x over previous
import jax
import jax.numpy as jnp
from jax.experimental import pallas as pl
from jax.experimental.pallas import tpu as pltpu


def _scores_kernel(q_ref, k_ref, s_ref, mm_ref, qs_ref):
    i = pl.program_id(1)

    @pl.when(i == 0)
    def _qsum():
        qs_ref[...] = jnp.sum(q_ref[0, 0], axis=-1, keepdims=True)

    prod = k_ref[0] * qs_ref[...][None]
    sb = jnp.sum(prod, axis=2)
    s_ref[0] = sb.astype(jnp.bfloat16)
    mn = jnp.min(sb, axis=(0, 2), keepdims=True)[0]
    mx = jnp.max(sb, axis=(0, 2), keepdims=True)[0]

    @pl.when(i == 0)
    def _init():
        mm_ref[0, 0] = mn
        mm_ref[0, 1] = mx

    @pl.when(i > 0)
    def _update():
        mm_ref[0, 0] = jnp.minimum(mm_ref[0, 0], mn)
        mm_ref[0, 1] = jnp.maximum(mm_ref[0, 1], mx)


def _weight_kernel(s_ref, mm_ref, v_ref, o_ref):
    mn = mm_ref[0, 0]
    mx = mm_ref[0, 1]
    inv = 1.0 / (mx - mn)
    attn = (s_ref[0].astype(jnp.float32) - mn) * inv
    sblk, h, e, p = v_ref.shape[1:]
    a4 = attn.reshape(sblk, h, 1, p)
    o_ref[0] = v_ref[0] * a4


@jax.jit
def kernel(queries, keys, values, attn_mask):
    del attn_mask
    B, L, Qd, H, E = queries.shape
    S, P = keys.shape[1], keys.shape[2]
    SA = 64
    NSA = S // SA
    SB = 32
    NSB = S // SB

    qt = jnp.transpose(queries, (0, 2, 3, 4, 1))
    kt = jnp.transpose(keys, (0, 1, 3, 4, 2))
    vt = jnp.transpose(values, (0, 1, 3, 4, 2))

    scores, mnmx = pl.pallas_call(
        _scores_kernel,
        grid=(B, NSA),
        in_specs=[
            pl.BlockSpec((1, 1, H, E, L), lambda b, i: (b, Qd - 1, 0, 0, 0)),
            pl.BlockSpec((1, SA, H, E, P), lambda b, i: (b, i, 0, 0, 0)),
        ],
        out_specs=[
            pl.BlockSpec((1, SA, H, P), lambda b, i: (b, i, 0, 0)),
            pl.BlockSpec((1, 2, H, 1), lambda b, i: (b, 0, 0, 0)),
        ],
        out_shape=[
            jax.ShapeDtypeStruct((B, S, H, P), jnp.bfloat16),
            jax.ShapeDtypeStruct((B, 2, H, 1), jnp.float32),
        ],
        scratch_shapes=[pltpu.VMEM((H, E, 1), jnp.float32)],
        compiler_params=pltpu.CompilerParams(
            dimension_semantics=("parallel", "arbitrary"),
            vmem_limit_bytes=56 * 1024 * 1024),
    )(qt, kt)

    out_t = pl.pallas_call(
        _weight_kernel,
        grid=(B, NSB),
        in_specs=[
            pl.BlockSpec((1, SB, H, P), lambda b, i: (b, i, 0, 0)),
            pl.BlockSpec((1, 2, H, 1), lambda b, i: (b, 0, 0, 0)),
            pl.BlockSpec((1, SB, H, E, P), lambda b, i: (b, i, 0, 0, 0)),
        ],
        out_specs=pl.BlockSpec((1, SB, H, E, P), lambda b, i: (b, i, 0, 0, 0)),
        out_shape=jax.ShapeDtypeStruct((B, S, H, E, P), jnp.float32),
        compiler_params=pltpu.CompilerParams(
            dimension_semantics=("parallel", "arbitrary"),
            vmem_limit_bytes=56 * 1024 * 1024),
    )(scores, mnmx, vt)

    return jnp.transpose(out_t, (0, 1, 4, 2, 3))
